# Initial kernel scaffold; baseline (speedup 1.0000x reference)
import jax
import jax.numpy as jnp
from jax.experimental import pallas as pl


def kernel(x, codebook):
    raise NotImplementedError("write your pallas kernel here")


# speedup vs baseline: 1.3153x; 1.3153x over previous
import functools

import jax
import jax.numpy as jnp
from jax import lax
from jax.experimental import pallas as pl
from jax.experimental.pallas import tpu as pltpu
from jax.experimental.pallas import tpu_sc as plsc

_COMMIT = 0.25
_E = 8192
_D = 32
_N = 32768
_TOK_BLK = 1024
_CODE_CHUNK = 2048
_NW = 32
_TPW = _N // _NW
_GCH = 128


def _argmin_body(x_ref, zsq_ref, esq_ref, cbt_ref, idx_ref):
    xb = x_ref[...]
    zsq = zsq_ref[...]
    accv = None
    acci = None
    for half in range(2):
        m_h = None
        a_h = None
        for cc in range(4096 // _CODE_CHUNK):
            c0 = half * 4096 + cc * _CODE_CHUNK
            sl = pl.ds(c0, _CODE_CHUNK)
            sc = lax.dot_general(xb, cbt_ref[:, sl],
                                 (((1,), (0,)), ((), ())),
                                 preferred_element_type=jnp.float32)
            dist = (zsq + esq_ref[:, sl]) - 2.0 * sc
            lm = jnp.min(dist, axis=1, keepdims=True)
            ii = lax.broadcasted_iota(jnp.int32, dist.shape, 1)
            la = jnp.min(jnp.where(dist == lm, ii, _E), axis=1,
                         keepdims=True) + c0
            if m_h is None:
                m_h, a_h = lm, la
            else:
                take = lm < m_h
                m_h = jnp.where(take, lm, m_h)
                a_h = jnp.where(take, la, a_h)
        if accv is None:
            accv, acci = m_h.astype(jnp.bfloat16), a_h
        else:
            better = m_h < accv.astype(jnp.float32)
            acci = jnp.where(better, a_h, acci)
    idx_ref[...] = acci


def _argmin_indices(xf, zsq, esq, cbt):
    out = pl.pallas_call(
        _argmin_body,
        grid=(_N // _TOK_BLK,),
        in_specs=[
            pl.BlockSpec((_TOK_BLK, _D), lambda i: (i, 0)),
            pl.BlockSpec((_TOK_BLK, 1), lambda i: (i, 0)),
            pl.BlockSpec((1, _E), lambda i: (0, 0)),
            pl.BlockSpec((_D, _E), lambda i: (0, 0)),
        ],
        out_specs=pl.BlockSpec((_TOK_BLK, 1), lambda i: (i, 0)),
        out_shape=jax.ShapeDtypeStruct((_N, 1), jnp.int32),
    )(xf, zsq, esq, cbt)
    return out


def _sc_lookup_body(cb_hbm, idx_hbm, x_hbm, out_hbm, part_hbm,
                    idx_v, rows_v, x_v, acc_v, sem):
    cid = lax.axis_index("c")
    sid = lax.axis_index("s")
    wid = sid * 2 + cid
    base = wid * _TPW
    nch = _TPW // _GCH
    pltpu.sync_copy(idx_hbm.at[pl.ds(wid * nch, nch)], idx_v)
    pltpu.sync_copy(x_hbm.at[pl.ds(base, _TPW)], x_v)
    copies = []
    for g in range(nch):
        copies.append(pltpu.async_copy(
            cb_hbm.at[idx_v.at[g]],
            rows_v.at[pl.ds(g * _GCH, _GCH)], sem))
    for cp in copies:
        cp.wait()

    def body(i, acc):
        for h in range(_D // 16):
            zq = rows_v[i, pl.ds(h * 16, 16)]
            xx = x_v[i, pl.ds(h * 16, 16)]
            d = zq - xx
            rows_v[i, pl.ds(h * 16, 16)] = xx + d
            acc = acc + d * d
        return acc

    acc = lax.fori_loop(0, _TPW, body, jnp.zeros((16,), jnp.float32))
    acc_v[...] = acc
    pltpu.sync_copy(rows_v, out_hbm.at[pl.ds(base, _TPW)])
    pltpu.sync_copy(acc_v, part_hbm.at[wid])


@functools.cache
def _sc_lookup():
    return pl.kernel(
        _sc_lookup_body,
        out_type=[
            jax.ShapeDtypeStruct((_N, _D), jnp.float32),
            jax.ShapeDtypeStruct((_NW, 16), jnp.float32),
        ],
        mesh=plsc.VectorSubcoreMesh(core_axis_name="c", subcore_axis_name="s"),
        compiler_params=pltpu.CompilerParams(use_tc_tiling_on_sc=False),
        scratch_types=[
            pltpu.VMEM((_TPW // _GCH, _GCH), jnp.int32),
            pltpu.VMEM((_TPW, _D), jnp.float32),
            pltpu.VMEM((_TPW, _D), jnp.float32),
            pltpu.VMEM((16,), jnp.float32),
            pltpu.SemaphoreType.DMA,
        ],
    )


def kernel(x, codebook):
    B, T, P, D = x.shape
    xf = x.reshape(B * T * P, D)
    zsq = jnp.sum(xf ** 2, axis=-1, keepdims=True)
    esq = jnp.sum(codebook ** 2, axis=-1).reshape(1, _E)
    cbt = codebook.T
    idx = _argmin_indices(xf, zsq, esq, cbt)
    idx2d = idx.reshape(_N // _GCH, _GCH)
    z_q_st_flat, partials = _sc_lookup()(codebook, idx2d, xf)
    m = jnp.sum(partials) / (B * T * P * D)
    loss = m + _COMMIT * m
    z_q_st = z_q_st_flat.reshape(B, T, P, D)
    return (z_q_st, loss, idx.reshape(-1))


# speedup vs baseline: 1.3491x; 1.0258x over previous
import functools

import jax
import jax.numpy as jnp
from jax import lax
from jax.experimental import pallas as pl
from jax.experimental.pallas import tpu as pltpu
from jax.experimental.pallas import tpu_sc as plsc

_COMMIT = 0.25
_E = 8192
_D = 32
_N = 32768
_TOK_BLK = 2048
_CODE_CHUNK = 2048
_NW = 32
_TPW = _N // _NW
_GCH = 128


def _argmin_body(x_ref, zsq_ref, esq_ref, cbt_ref, idx_ref):
    xb = x_ref[...]
    zsq = zsq_ref[...]
    accv = None
    acci = None
    for half in range(2):
        m_h = None
        a_h = None
        for cc in range(4096 // _CODE_CHUNK):
            c0 = half * 4096 + cc * _CODE_CHUNK
            sl = pl.ds(c0, _CODE_CHUNK)
            sc = lax.dot_general(xb, cbt_ref[:, sl],
                                 (((1,), (0,)), ((), ())),
                                 preferred_element_type=jnp.float32)
            dist = (zsq + esq_ref[:, sl]) - 2.0 * sc
            lm = jnp.min(dist, axis=1, keepdims=True)
            ii = lax.broadcasted_iota(jnp.int32, dist.shape, 1)
            la = jnp.min(jnp.where(dist == lm, ii, _E), axis=1,
                         keepdims=True) + c0
            if m_h is None:
                m_h, a_h = lm, la
            else:
                take = lm < m_h
                m_h = jnp.where(take, lm, m_h)
                a_h = jnp.where(take, la, a_h)
        if accv is None:
            accv, acci = m_h.astype(jnp.bfloat16), a_h
        else:
            better = m_h < accv.astype(jnp.float32)
            acci = jnp.where(better, a_h, acci)
    idx_ref[...] = acci


def _argmin_indices(xf, zsq, esq, cbt):
    out = pl.pallas_call(
        _argmin_body,
        grid=(_N // _TOK_BLK,),
        in_specs=[
            pl.BlockSpec((_TOK_BLK, _D), lambda i: (i, 0)),
            pl.BlockSpec((_TOK_BLK, 1), lambda i: (i, 0)),
            pl.BlockSpec((1, _E), lambda i: (0, 0)),
            pl.BlockSpec((_D, _E), lambda i: (0, 0)),
        ],
        out_specs=pl.BlockSpec((_TOK_BLK, 1), lambda i: (i, 0)),
        out_shape=jax.ShapeDtypeStruct((_N, 1), jnp.int32),
    )(xf, zsq, esq, cbt)
    return out


def _sc_lookup_body(cb_hbm, idx_hbm, x_hbm, out_hbm, part_hbm,
                    idx_v, rows_v, x_v, acc_v, sem):
    cid = lax.axis_index("c")
    sid = lax.axis_index("s")
    wid = sid * 2 + cid
    base = wid * _TPW
    nch = _TPW // _GCH
    pltpu.sync_copy(idx_hbm.at[pl.ds(wid * nch, nch)], idx_v)
    pltpu.sync_copy(x_hbm.at[pl.ds(base, _TPW)], x_v)
    copies = []
    for g in range(nch):
        copies.append(pltpu.async_copy(
            cb_hbm.at[idx_v.at[g]],
            rows_v.at[pl.ds(g * _GCH, _GCH)], sem))
    for cp in copies:
        cp.wait()

    def body(i, acc):
        for h in range(_D // 16):
            zq = rows_v[i, pl.ds(h * 16, 16)]
            xx = x_v[i, pl.ds(h * 16, 16)]
            d = zq - xx
            rows_v[i, pl.ds(h * 16, 16)] = xx + d
            acc = acc + d * d
        return acc

    acc = lax.fori_loop(0, _TPW, body, jnp.zeros((16,), jnp.float32))
    acc_v[...] = acc
    pltpu.sync_copy(rows_v, out_hbm.at[pl.ds(base, _TPW)])
    pltpu.sync_copy(acc_v, part_hbm.at[wid])


@functools.cache
def _sc_lookup():
    return pl.kernel(
        _sc_lookup_body,
        out_type=[
            jax.ShapeDtypeStruct((_N, _D), jnp.float32),
            jax.ShapeDtypeStruct((_NW, 16), jnp.float32),
        ],
        mesh=plsc.VectorSubcoreMesh(core_axis_name="c", subcore_axis_name="s"),
        compiler_params=pltpu.CompilerParams(use_tc_tiling_on_sc=False),
        scratch_types=[
            pltpu.VMEM((_TPW // _GCH, _GCH), jnp.int32),
            pltpu.VMEM((_TPW, _D), jnp.float32),
            pltpu.VMEM((_TPW, _D), jnp.float32),
            pltpu.VMEM((16,), jnp.float32),
            pltpu.SemaphoreType.DMA,
        ],
    )


def kernel(x, codebook):
    B, T, P, D = x.shape
    xf = x.reshape(B * T * P, D)
    zsq = jnp.sum(xf ** 2, axis=-1, keepdims=True)
    esq = jnp.sum(codebook ** 2, axis=-1).reshape(1, _E)
    cbt = codebook.T
    idx = _argmin_indices(xf, zsq, esq, cbt)
    idx2d = idx.reshape(_N // _GCH, _GCH)
    z_q_st_flat, partials = _sc_lookup()(codebook, idx2d, xf)
    m = jnp.sum(partials) / (B * T * P * D)
    loss = m + _COMMIT * m
    z_q_st = z_q_st_flat.reshape(B, T, P, D)
    return (z_q_st, loss, idx.reshape(-1))
